# baseline (speedup 1.0000x reference)
import functools
import jax
import jax.numpy as jnp
from jax import lax
from jax.experimental import pallas as pl
from jax.experimental.pallas import tpu as pltpu

NNODES = 10000
NREL = 16
NEMB = 64
R_TOTAL = 2 * NREL + 1
N_EDGES = 320000
N_AUG = 2 * N_EDGES
N_TRIPLES = 16384


def _relu_matmul_kernel(x_ref, b_ref, w_ref, y_ref):
    x = jnp.maximum(x_ref[...] + b_ref[...], 0.0)
    y_ref[0] = jnp.dot(x, w_ref[0], preferred_element_type=jnp.float32)


def _tc_relu_matmul(x, bias, W):
    R = W.shape[0]
    return pl.pallas_call(
        _relu_matmul_kernel,
        grid=(R,),
        in_specs=[
            pl.BlockSpec((NNODES, NEMB), lambda r: (0, 0)),
            pl.BlockSpec((1, NEMB), lambda r: (0, 0)),
            pl.BlockSpec((1, NEMB, NEMB), lambda r: (r, 0, 0)),
        ],
        out_specs=pl.BlockSpec((1, NNODES, NEMB), lambda r: (r, 0, 0)),
        out_shape=jax.ShapeDtypeStruct((R, NNODES, NEMB), jnp.float32),
    )(x, bias, W)


def kernel(node_embeddings, node_embeddings_bias, W1, b1, W2, b2, relations, graph, triples):
    s = graph[:, 0]
    r = graph[:, 1] % NREL
    o = graph[:, 2]
    s_aug = jnp.concatenate([s, o])
    o_aug = jnp.concatenate([o, s])
    r_aug = jnp.concatenate([r, r + NREL])

    seg = r_aug * NNODES + o_aug
    gidx = r_aug * NNODES + s_aug
    ones = jnp.ones((N_AUG,), jnp.float32)
    counts = jax.ops.segment_sum(ones, seg, num_segments=2 * NREL * NNODES)
    norm = 1.0 / jnp.maximum(counts[seg], 1.0)

    y1 = _tc_relu_matmul(node_embeddings, node_embeddings_bias, W1)
    y1f = y1.reshape(R_TOTAL * NNODES, NEMB)
    msg1 = y1f[gidx] * norm[:, None]
    h1pre = jax.ops.segment_sum(msg1, o_aug, num_segments=NNODES)
    h1pre = h1pre + y1[2 * NREL] + b1

    y2 = _tc_relu_matmul(h1pre, jnp.zeros((1, NEMB), jnp.float32), W2)
    y2f = y2.reshape(R_TOTAL * NNODES, NEMB)
    msg2 = y2f[gidx] * norm[:, None]
    h2 = jax.ops.segment_sum(msg2, o_aug, num_segments=NNODES)
    h2 = h2 + y2[2 * NREL] + b2

    ts = triples[:, 0]
    tp = triples[:, 1] % NREL
    to = triples[:, 2]
    scores = jnp.sum(h2[ts] * relations[tp] * h2[to], axis=-1)
    penalty = jnp.sum(relations ** 2)
    return (scores, penalty)


# speedup vs baseline: 7.3538x; 7.3538x over previous
import functools
import jax
import jax.numpy as jnp
from jax import lax
from jax.experimental import pallas as pl
from jax.experimental.pallas import tpu as pltpu
from jax.experimental.pallas import tpu_sc as plsc

NNODES = 10000
NREL = 16
NEMB = 64
R_TOTAL = 2 * NREL + 1
N_EDGES = 320000
N_TRIPLES = 16384

NC, NS, L = 2, 16, 16
NW = NC * NS
CHUNK = 128
N_CHUNKS = (2 * N_EDGES) // CHUNK
FWD_CHUNKS = N_EDGES // CHUNK
NSEG = 2 * NREL * NNODES
NSEG_HALF = NREL * NNODES
SUB = CHUNK // L

_MESH = plsc.VectorSubcoreMesh(core_axis_name="c", subcore_axis_name="s")
_SC_PARAMS = pltpu.CompilerParams(use_tc_tiling_on_sc=False, needs_layout_passes=False)


def _split_chunks(total, parts, i):
    base = total // parts
    rem = total - base * parts
    start = i * base + jnp.minimum(i, rem)
    extra = jnp.where(i < rem, 1, 0)
    return start, start + base + extra


def _k_a_body(x_ref, b_ref, w_ref, rel_ref, y_ref, pen_ref):
    x = jnp.maximum(x_ref[...] + b_ref[...], 0.0)
    y_ref[0] = jnp.dot(x, w_ref[0], preferred_element_type=jnp.float32)

    @pl.when(pl.program_id(0) == 0)
    def _():
        pen_ref[...] = jnp.sum(rel_ref[...] ** 2).reshape(1, 1)


def _tc_layer1(ne, bias, W1, relations):
    return pl.pallas_call(
        _k_a_body,
        grid=(R_TOTAL,),
        in_specs=[
            pl.BlockSpec((NNODES, NEMB), lambda r: (0, 0)),
            pl.BlockSpec((1, NEMB), lambda r: (0, 0)),
            pl.BlockSpec((1, NEMB, NEMB), lambda r: (r, 0, 0)),
            pl.BlockSpec((NREL, NEMB), lambda r: (0, 0)),
        ],
        out_specs=[
            pl.BlockSpec((1, NNODES, NEMB), lambda r: (r, 0, 0)),
            pl.BlockSpec((1, 1), lambda r: (0, 0)),
        ],
        out_shape=[
            jax.ShapeDtypeStruct((R_TOTAL, NNODES, NEMB), jnp.float32),
            jax.ShapeDtypeStruct((1, 1), jnp.float32),
        ],
    )(ne, bias, W1, relations)


def _k_e_body(p_ref, yself_ref, b_ref, w_ref, y_ref):
    x = p_ref[0] + p_ref[1] + yself_ref[0] + b_ref[...]
    x = jnp.maximum(x, 0.0)
    y_ref[0] = jnp.dot(x, w_ref[0], preferred_element_type=jnp.float32)


def _tc_layer2(hpart, y1, b1, W2):
    return pl.pallas_call(
        _k_e_body,
        grid=(R_TOTAL,),
        in_specs=[
            pl.BlockSpec((2, NNODES, NEMB), lambda r: (0, 0, 0)),
            pl.BlockSpec((1, NNODES, NEMB), lambda r: (R_TOTAL - 1, 0, 0)),
            pl.BlockSpec((1, NEMB), lambda r: (0, 0)),
            pl.BlockSpec((1, NEMB, NEMB), lambda r: (r, 0, 0)),
        ],
        out_specs=pl.BlockSpec((1, NNODES, NEMB), lambda r: (r, 0, 0)),
        out_shape=jax.ShapeDtypeStruct((R_TOTAL, NNODES, NEMB), jnp.float32),
    )(hpart, y1, b1.reshape(1, NEMB), W2)


def _k_f_body(p_ref, yself_ref, b_ref, h_ref):
    h_ref[...] = p_ref[0] + p_ref[1] + yself_ref[0] + b_ref[...]


def _tc_final(hpart, y2, b2):
    return pl.pallas_call(
        _k_f_body,
        grid=(1,),
        in_specs=[
            pl.BlockSpec((2, NNODES, NEMB), lambda i: (0, 0, 0)),
            pl.BlockSpec((1, NNODES, NEMB), lambda i: (R_TOTAL - 1, 0, 0)),
            pl.BlockSpec((1, NEMB), lambda i: (0, 0)),
        ],
        out_specs=pl.BlockSpec((NNODES, NEMB), lambda i: (0, 0)),
        out_shape=jax.ShapeDtypeStruct((NNODES, NEMB), jnp.float32),
    )(hpart, y2, b2.reshape(1, NEMB))


def _k_counts_body(s_hbm, r_hbm, o_hbm, counts_hbm,
                   rbuf, dbuf, segbuf, onesbuf, zbuf, counts_sh, sem):
    cid = lax.axis_index("c")
    sid = lax.axis_index("s")

    def _init(i, _):
        onesbuf[0, pl.ds(i * L, L)] = jnp.full((L,), 1.0, jnp.float32)
        return 0
    lax.fori_loop(0, CHUNK // L, _init, 0)

    def _zero(i, _):
        zbuf[pl.ds(i * L, L)] = jnp.zeros((L,), jnp.float32)
        return 0
    lax.fori_loop(0, 2000 // L, _zero, 0)

    def _zslice(i, _):
        pltpu.sync_copy(zbuf, counts_sh.at[pl.ds(sid * 10000 + i * 2000, 2000)])
        return 0
    lax.fori_loop(0, 5, _zslice, 0)
    plsc.subcore_barrier()

    c0, c1 = _split_chunks(FWD_CHUNKS, NS, sid)
    @pl.when(cid == 0)
    def _fwd():
        def _chunkf(ci, _):
            eoff = ci * CHUNK
            pltpu.sync_copy(r_hbm.at[pl.ds(eoff, CHUNK)], rbuf)
            pltpu.sync_copy(o_hbm.at[pl.ds(eoff, CHUNK)], dbuf)
            def _seg(j, _):
                rel = lax.bitwise_and(rbuf[pl.ds(j * L, L)], NREL - 1)
                seg = rel * NNODES + dbuf[pl.ds(j * L, L)]
                segbuf[0, pl.ds(j * L, L)] = seg
                return 0
            lax.fori_loop(0, SUB, _seg, 0)
            pltpu.sync_copy(onesbuf.at[0], counts_sh.at[segbuf.at[0]], add=True)
            return 0
        lax.fori_loop(c0, c1, _chunkf, 0)

    @pl.when(cid == 1)
    def _inv():
        def _chunki(ci, _):
            eoff = ci * CHUNK
            pltpu.sync_copy(r_hbm.at[pl.ds(eoff, CHUNK)], rbuf)
            pltpu.sync_copy(s_hbm.at[pl.ds(eoff, CHUNK)], dbuf)
            def _seg(j, _):
                rel = lax.bitwise_and(rbuf[pl.ds(j * L, L)], NREL - 1)
                seg = rel * NNODES + dbuf[pl.ds(j * L, L)]
                segbuf[0, pl.ds(j * L, L)] = seg
                return 0
            lax.fori_loop(0, SUB, _seg, 0)
            pltpu.sync_copy(onesbuf.at[0], counts_sh.at[segbuf.at[0]], add=True)
            return 0
        lax.fori_loop(c0, c1, _chunki, 0)

    plsc.subcore_barrier()
    def _wslice(i, _):
        pltpu.sync_copy(counts_sh.at[pl.ds(sid * 10000 + i * 2000, 2000)], zbuf)
        pltpu.sync_copy(
            zbuf, counts_hbm.at[pl.ds(cid * NSEG_HALF + sid * 10000 + i * 2000, 2000)])
        return 0
    lax.fori_loop(0, 5, _wslice, 0)


def _sc_counts(s_col, r_col, o_col):
    k = pl.kernel(
        _k_counts_body,
        out_type=jax.ShapeDtypeStruct((NSEG,), jnp.float32),
        mesh=_MESH,
        compiler_params=_SC_PARAMS,
        scratch_types=[
            pltpu.VMEM((CHUNK,), jnp.int32),
            pltpu.VMEM((CHUNK,), jnp.int32),
            pltpu.VMEM((1, CHUNK), jnp.int32),
            pltpu.VMEM((1, CHUNK), jnp.float32),
            pltpu.VMEM((2000,), jnp.float32),
            pltpu.VMEM_SHARED((NSEG_HALF,), jnp.float32),
            pltpu.SemaphoreType.DMA,
        ],
    )
    return k(s_col, r_col, o_col)


def _k_layer_body(y_hbm, s_hbm, r_hbm, o_hbm, counts_hbm, hpart_hbm,
                  rbuf, sbuf, obuf, gidxbuf, dstbuf, cntbuf, nrmbuf,
                  rowbuf, zbuf, acc_sh, sem):
    cid = lax.axis_index("c")
    sid = lax.axis_index("s")
    wid = cid * NS + sid

    def _z(i, _):
        zbuf[i, pl.ds(0, L)] = jnp.zeros((L,), jnp.float32)
        zbuf[i, pl.ds(L, L)] = jnp.zeros((L,), jnp.float32)
        zbuf[i, pl.ds(2 * L, L)] = jnp.zeros((L,), jnp.float32)
        zbuf[i, pl.ds(3 * L, L)] = jnp.zeros((L,), jnp.float32)
        return 0
    lax.fori_loop(0, 125, _z, 0)

    def _zslice(i, _):
        pltpu.sync_copy(zbuf, acc_sh.at[pl.ds(sid * 625 + i * 125, 125)])
        return 0
    lax.fori_loop(0, 5, _zslice, 0)
    plsc.subcore_barrier()

    c0, c1 = _split_chunks(N_CHUNKS, NW, wid)

    def _chunk(ci, _):
        eoff = (ci % FWD_CHUNKS) * CHUNK
        inv = (ci >= FWD_CHUNKS).astype(jnp.int32)
        pltpu.sync_copy(r_hbm.at[pl.ds(eoff, CHUNK)], rbuf)
        pltpu.sync_copy(s_hbm.at[pl.ds(eoff, CHUNK)], sbuf)
        pltpu.sync_copy(o_hbm.at[pl.ds(eoff, CHUNK)], obuf)

        def _idx(j, _):
            r16 = lax.bitwise_and(rbuf[pl.ds(j * L, L)], NREL - 1)
            sv = sbuf[pl.ds(j * L, L)]
            ov = obuf[pl.ds(j * L, L)]
            src = jnp.where(inv == 1, ov, sv)
            dst = jnp.where(inv == 1, sv, ov)
            rel = r16 + inv * NREL
            gidxbuf[0, pl.ds(j * L, L)] = rel * NNODES + src
            dstbuf[0, pl.ds(j * L, L)] = dst
            cntidx = rel * NNODES + dst
            gidxbuf[1, pl.ds(j * L, L)] = cntidx
            return 0
        lax.fori_loop(0, SUB, _idx, 0)

        pltpu.async_copy(counts_hbm.at[gidxbuf.at[1]], cntbuf, sem).wait()
        def _nrm(j, _):
            c = cntbuf[pl.ds(j * L, L)]
            nrmbuf[0, pl.ds(j * L, L)] = 1.0 / jnp.maximum(c, 1.0)
            return 0
        lax.fori_loop(0, SUB, _nrm, 0)

        pltpu.async_copy(y_hbm.at[gidxbuf.at[0]], rowbuf, sem).wait()

        def _scale(e, _):
            nv = plsc.load_gather(
                nrmbuf, [jnp.zeros((L,), jnp.int32), jnp.full((L,), e, jnp.int32)])
            rowbuf[e, pl.ds(0, L)] = rowbuf[e, pl.ds(0, L)] * nv
            rowbuf[e, pl.ds(L, L)] = rowbuf[e, pl.ds(L, L)] * nv
            rowbuf[e, pl.ds(2 * L, L)] = rowbuf[e, pl.ds(2 * L, L)] * nv
            rowbuf[e, pl.ds(3 * L, L)] = rowbuf[e, pl.ds(3 * L, L)] * nv
            return 0
        lax.fori_loop(0, CHUNK, _scale, 0)

        pltpu.sync_copy(rowbuf, acc_sh.at[dstbuf.at[0]], add=True)
        return 0

    lax.fori_loop(c0, c1, _chunk, 0)
    plsc.subcore_barrier()

    def _out(i, _):
        pltpu.sync_copy(acc_sh.at[pl.ds(sid * 625 + i * 125, 125)], zbuf)
        pltpu.sync_copy(zbuf, hpart_hbm.at[cid, pl.ds(sid * 625 + i * 125, 125)])
        return 0
    lax.fori_loop(0, 5, _out, 0)


def _sc_layer(yf, s_col, r_col, o_col, counts):
    k = pl.kernel(
        _k_layer_body,
        out_type=jax.ShapeDtypeStruct((2, NNODES, NEMB), jnp.float32),
        mesh=_MESH,
        compiler_params=_SC_PARAMS,
        scratch_types=[
            pltpu.VMEM((CHUNK,), jnp.int32),
            pltpu.VMEM((CHUNK,), jnp.int32),
            pltpu.VMEM((CHUNK,), jnp.int32),
            pltpu.VMEM((2, CHUNK), jnp.int32),
            pltpu.VMEM((1, CHUNK), jnp.int32),
            pltpu.VMEM((CHUNK,), jnp.float32),
            pltpu.VMEM((1, CHUNK), jnp.float32),
            pltpu.VMEM((CHUNK, NEMB), jnp.float32),
            pltpu.VMEM((125, NEMB), jnp.float32),
            pltpu.VMEM_SHARED((NNODES, NEMB), jnp.float32),
            pltpu.SemaphoreType.DMA,
        ],
    )
    return k(yf, s_col, r_col, o_col, counts)


def _k_distmult_body(h_hbm, rel_hbm, ts_hbm, tp_hbm, to_hbm, scores_hbm,
                     tsbuf, tpbuf, tobuf, idxbuf, hsrow, horow, relrow,
                     tmpbuf, scorebuf, sem):
    cid = lax.axis_index("c")
    sid = lax.axis_index("s")
    wid = cid * NS + sid
    per_w = N_TRIPLES // NW
    n_ch = per_w // CHUNK

    def _chunk(ci, _):
        toff = wid * per_w + ci * CHUNK
        pltpu.sync_copy(ts_hbm.at[pl.ds(toff, CHUNK)], tsbuf)
        pltpu.sync_copy(tp_hbm.at[pl.ds(toff, CHUNK)], tpbuf)
        pltpu.sync_copy(to_hbm.at[pl.ds(toff, CHUNK)], tobuf)

        def _idx(j, _):
            idxbuf[0, pl.ds(j * L, L)] = tsbuf[pl.ds(j * L, L)]
            idxbuf[1, pl.ds(j * L, L)] = lax.bitwise_and(tpbuf[pl.ds(j * L, L)], NREL - 1)
            idxbuf[2, pl.ds(j * L, L)] = tobuf[pl.ds(j * L, L)]
            return 0
        lax.fori_loop(0, SUB, _idx, 0)

        pltpu.async_copy(h_hbm.at[idxbuf.at[0]], hsrow, sem).wait()
        pltpu.async_copy(rel_hbm.at[idxbuf.at[1]], relrow, sem).wait()
        pltpu.async_copy(h_hbm.at[idxbuf.at[2]], horow, sem).wait()

        def _dot(e, _):
            acc = (hsrow[e, pl.ds(0, L)] * relrow[e, pl.ds(0, L)] * horow[e, pl.ds(0, L)]
                   + hsrow[e, pl.ds(L, L)] * relrow[e, pl.ds(L, L)] * horow[e, pl.ds(L, L)]
                   + hsrow[e, pl.ds(2 * L, L)] * relrow[e, pl.ds(2 * L, L)] * horow[e, pl.ds(2 * L, L)]
                   + hsrow[e, pl.ds(3 * L, L)] * relrow[e, pl.ds(3 * L, L)] * horow[e, pl.ds(3 * L, L)])
            tmpbuf[e, pl.ds(0, L)] = acc
            return 0
        lax.fori_loop(0, CHUNK, _dot, 0)

        rows16 = lax.iota(jnp.int32, L)

        def _red(j, _):
            ridx = rows16 + j * L
            sv = jnp.zeros((L,), jnp.float32)
            def _acc(k, sv):
                return sv + plsc.load_gather(tmpbuf, [ridx, jnp.full((L,), k, jnp.int32)])
            sv = lax.fori_loop(0, L, _acc, sv)
            scorebuf[pl.ds(j * L, L)] = sv
            return 0
        lax.fori_loop(0, SUB, _red, 0)

        pltpu.sync_copy(scorebuf, scores_hbm.at[pl.ds(toff, CHUNK)])
        return 0

    lax.fori_loop(0, n_ch, _chunk, 0)


def _sc_distmult(h2, relations, ts, tp, to):
    k = pl.kernel(
        _k_distmult_body,
        out_type=jax.ShapeDtypeStruct((N_TRIPLES,), jnp.float32),
        mesh=_MESH,
        compiler_params=_SC_PARAMS,
        scratch_types=[
            pltpu.VMEM((CHUNK,), jnp.int32),
            pltpu.VMEM((CHUNK,), jnp.int32),
            pltpu.VMEM((CHUNK,), jnp.int32),
            pltpu.VMEM((3, CHUNK), jnp.int32),
            pltpu.VMEM((CHUNK, NEMB), jnp.float32),
            pltpu.VMEM((CHUNK, NEMB), jnp.float32),
            pltpu.VMEM((CHUNK, NEMB), jnp.float32),
            pltpu.VMEM((CHUNK, L), jnp.float32),
            pltpu.VMEM((CHUNK,), jnp.float32),
            pltpu.SemaphoreType.DMA,
        ],
    )
    return k(h2, relations, ts, tp, to)


def kernel(node_embeddings, node_embeddings_bias, W1, b1, W2, b2, relations, graph, triples):
    s_col = graph[:, 0]
    r_col = graph[:, 1]
    o_col = graph[:, 2]
    ts = triples[:, 0]
    tp = triples[:, 1]
    to = triples[:, 2]

    y1, pen = _tc_layer1(node_embeddings, node_embeddings_bias, W1, relations)
    counts = _sc_counts(s_col, r_col, o_col)

    y1f = y1.reshape(R_TOTAL * NNODES, NEMB)
    hpart1 = _sc_layer(y1f, s_col, r_col, o_col, counts)

    y2 = _tc_layer2(hpart1, y1, b1, W2)
    y2f = y2.reshape(R_TOTAL * NNODES, NEMB)
    hpart2 = _sc_layer(y2f, s_col, r_col, o_col, counts)

    h2 = _tc_final(hpart2, y2, b2)
    scores = _sc_distmult(h2, relations, ts, tp, to)
    return (scores, pen[0, 0])


# speedup vs baseline: 11.0395x; 1.5012x over previous
import functools
import jax
import jax.numpy as jnp
from jax import lax
from jax.experimental import pallas as pl
from jax.experimental.pallas import tpu as pltpu
from jax.experimental.pallas import tpu_sc as plsc

NNODES = 10000
NREL = 16
NEMB = 64
R_TOTAL = 2 * NREL + 1
N_EDGES = 320000
N_TRIPLES = 16384

NC, NS, L = 2, 16, 16
NW = NC * NS
CHUNK = 128
PAD_EDGES = 327680
FWD_CHUNKS = PAD_EDGES // CHUNK
N_CHUNKS = 2 * FWD_CHUNKS
CPW = N_CHUNKS // NW
CPT = FWD_CHUNKS // NS
MAC = 16
NSEG = 2 * NREL * NNODES + 16
NSEG_HALF = NREL * NNODES
ACC_ROWS = NNODES + 16
SUB = CHUNK // L

_MESH = plsc.VectorSubcoreMesh(core_axis_name="c", subcore_axis_name="s")
_SC_PARAMS = pltpu.CompilerParams(use_tc_tiling_on_sc=False, needs_layout_passes=False)


def _k_a_body(x_ref, b_ref, w_ref, rel_ref, y_ref, pen_ref):
    x = jnp.maximum(x_ref[...] + b_ref[...], 0.0)
    y_ref[0] = jnp.dot(x, w_ref[0], preferred_element_type=jnp.float32)

    @pl.when(pl.program_id(0) == 0)
    def _():
        pen_ref[...] = jnp.sum(rel_ref[...] ** 2).reshape(1, 1)


def _tc_layer1(ne, bias, W1, relations):
    return pl.pallas_call(
        _k_a_body,
        grid=(R_TOTAL,),
        in_specs=[
            pl.BlockSpec((NNODES, NEMB), lambda r: (0, 0)),
            pl.BlockSpec((1, NEMB), lambda r: (0, 0)),
            pl.BlockSpec((1, NEMB, NEMB), lambda r: (r, 0, 0)),
            pl.BlockSpec((NREL, NEMB), lambda r: (0, 0)),
        ],
        out_specs=[
            pl.BlockSpec((1, NNODES, NEMB), lambda r: (r, 0, 0)),
            pl.BlockSpec((1, 1), lambda r: (0, 0)),
        ],
        out_shape=[
            jax.ShapeDtypeStruct((R_TOTAL, NNODES, NEMB), jnp.float32),
            jax.ShapeDtypeStruct((1, 1), jnp.float32),
        ],
    )(ne, bias, W1, relations)


def _k_e_body(p_ref, yself_ref, b_ref, w_ref, y_ref):
    x = p_ref[0] + p_ref[1] + yself_ref[0] + b_ref[...]
    x = jnp.maximum(x, 0.0)
    y_ref[0] = jnp.dot(x, w_ref[0], preferred_element_type=jnp.float32)


def _tc_layer2(hpart, y1, b1, W2):
    return pl.pallas_call(
        _k_e_body,
        grid=(R_TOTAL,),
        in_specs=[
            pl.BlockSpec((2, NNODES, NEMB), lambda r: (0, 0, 0)),
            pl.BlockSpec((1, NNODES, NEMB), lambda r: (R_TOTAL - 1, 0, 0)),
            pl.BlockSpec((1, NEMB), lambda r: (0, 0)),
            pl.BlockSpec((1, NEMB, NEMB), lambda r: (r, 0, 0)),
        ],
        out_specs=pl.BlockSpec((1, NNODES, NEMB), lambda r: (r, 0, 0)),
        out_shape=jax.ShapeDtypeStruct((R_TOTAL, NNODES, NEMB), jnp.float32),
    )(hpart, y1, b1.reshape(1, NEMB), W2)


def _k_f_body(p_ref, yself_ref, b_ref, h_ref):
    h_ref[...] = p_ref[0] + p_ref[1] + yself_ref[0] + b_ref[...]


def _tc_final(hpart, y2, b2):
    return pl.pallas_call(
        _k_f_body,
        grid=(1,),
        in_specs=[
            pl.BlockSpec((2, NNODES, NEMB), lambda i: (0, 0, 0)),
            pl.BlockSpec((1, NNODES, NEMB), lambda i: (R_TOTAL - 1, 0, 0)),
            pl.BlockSpec((1, NEMB), lambda i: (0, 0)),
        ],
        out_specs=pl.BlockSpec((NNODES, NEMB), lambda i: (0, 0)),
        out_shape=jax.ShapeDtypeStruct((NNODES, NEMB), jnp.float32),
    )(hpart, y2, b2.reshape(1, NEMB))


def _k_counts_body(s_hbm, r_hbm, o_hbm, counts_hbm,
                   rmac, dmac, segmac, onesbuf, zbuf, counts_sh, semL, semS):
    cid = lax.axis_index("c")
    sid = lax.axis_index("s")

    for i in range(SUB):
        onesbuf[0, pl.ds(i * L, L)] = jnp.full((L,), 1.0, jnp.float32)

    def _zero(i, _):
        zbuf[pl.ds(i * L, L)] = jnp.zeros((L,), jnp.float32)
        return 0
    lax.fori_loop(0, 2000 // L, _zero, 0)

    def _zslice(i, _):
        pltpu.sync_copy(zbuf, counts_sh.at[pl.ds(sid * 10000 + i * 2000, 2000)])
        return 0
    lax.fori_loop(0, 5, _zslice, 0)

    @pl.when(sid == 15)
    def _ztail():
        pltpu.sync_copy(zbuf.at[pl.ds(0, 16)], counts_sh.at[pl.ds(NSEG_HALF, 16)])
    plsc.subcore_barrier()

    def _macro(m, _):
        eoff = (sid * CPT + m * MAC) * CHUNK
        a = pltpu.async_copy(r_hbm.at[pl.ds(eoff, MAC * CHUNK)], rmac, semL)

        @pl.when(cid == 0)
        def _():
            pltpu.async_copy(o_hbm.at[pl.ds(eoff, MAC * CHUNK)], dmac, semL)

        @pl.when(cid == 1)
        def _():
            pltpu.async_copy(s_hbm.at[pl.ds(eoff, MAC * CHUNK)], dmac, semL)
        a.wait()
        pltpu.make_async_copy(o_hbm.at[pl.ds(0, MAC * CHUNK)], dmac, semL).wait()

        for k in range(MAC):
            for j in range(SUB):
                off = k * CHUNK + j * L
                rel = lax.bitwise_and(rmac[pl.ds(off, L)], NREL - 1)
                seg = rel * NNODES + dmac[pl.ds(off, L)]
                segmac[k, pl.ds(j * L, L)] = seg
            pltpu.async_copy(onesbuf.at[0], counts_sh.at[segmac.at[k]], semS, add=True)
        for k in range(MAC):
            pltpu.make_async_copy(onesbuf.at[0], counts_sh.at[segmac.at[k]], semS).wait()
        return 0
    lax.fori_loop(0, CPT // MAC, _macro, 0)

    plsc.subcore_barrier()
    def _wslice(i, _):
        pltpu.sync_copy(counts_sh.at[pl.ds(sid * 10000 + i * 2000, 2000)], zbuf)
        pltpu.sync_copy(
            zbuf, counts_hbm.at[pl.ds(cid * NSEG_HALF + sid * 10000 + i * 2000, 2000)])
        return 0
    lax.fori_loop(0, 5, _wslice, 0)

    @pl.when(jnp.logical_and(cid == 1, sid == 15))
    def _wtail():
        pltpu.sync_copy(counts_sh.at[pl.ds(NSEG_HALF, 16)], zbuf.at[pl.ds(0, 16)])
        pltpu.sync_copy(zbuf.at[pl.ds(0, 16)], counts_hbm.at[pl.ds(2 * NSEG_HALF, 16)])


def _sc_counts(s_col, r_col, o_col):
    k = pl.kernel(
        _k_counts_body,
        out_type=jax.ShapeDtypeStruct((NSEG,), jnp.float32),
        mesh=_MESH,
        compiler_params=_SC_PARAMS,
        scratch_types=[
            pltpu.VMEM((MAC * CHUNK,), jnp.int32),
            pltpu.VMEM((MAC * CHUNK,), jnp.int32),
            pltpu.VMEM((MAC, CHUNK), jnp.int32),
            pltpu.VMEM((1, CHUNK), jnp.float32),
            pltpu.VMEM((2000,), jnp.float32),
            pltpu.VMEM_SHARED((NSEG_HALF + 16,), jnp.float32),
            pltpu.SemaphoreType.DMA,
            pltpu.SemaphoreType.DMA,
        ],
    )
    return k(s_col, r_col, o_col)


def _k_layer_body(y_hbm, s_hbm, r_hbm, o_hbm, counts_hbm, hpart_hbm,
                  rbuf0, rbuf1, sbuf0, sbuf1, obuf0, obuf1,
                  gidx0, gidx1, dst0, dst1, cnt0, cnt1, nrm0, nrm1,
                  row0, row1, zbuf, acc_sh,
                  semrso0, semrso1, semcnt0, semcnt1,
                  semrow0, semrow1, semsca0, semsca1):
    cid = lax.axis_index("c")
    sid = lax.axis_index("s")
    wid = cid * NS + sid

    rbuf = (rbuf0, rbuf1)
    sbuf = (sbuf0, sbuf1)
    obuf = (obuf0, obuf1)
    gidx = (gidx0, gidx1)
    dstb = (dst0, dst1)
    cntb = (cnt0, cnt1)
    nrmb = (nrm0, nrm1)
    rowb = (row0, row1)
    semrso = (semrso0, semrso1)
    semcnt = (semcnt0, semcnt1)
    semrow = (semrow0, semrow1)
    semsca = (semsca0, semsca1)

    def _zfill(i, _):
        r = i // 4
        c = i % 4
        zbuf[r, pl.ds(c * L, L)] = jnp.zeros((L,), jnp.float32)
        return 0
    lax.fori_loop(0, 320, _zfill, 0)

    base_row = sid * 626

    def _zslice(i, _):
        pltpu.sync_copy(zbuf, acc_sh.at[pl.ds(base_row + i * 80, 80)])
        return 0
    lax.fori_loop(0, 7, _zslice, 0)
    pltpu.sync_copy(zbuf.at[pl.ds(0, 66)], acc_sh.at[pl.ds(base_row + 560, 66)])
    plsc.subcore_barrier()

    cbase = wid * CPW
    NCH = CPW

    def _issue_rso(j, p):
        eoff = (lax.rem(cbase + j, FWD_CHUNKS)) * CHUNK
        a = pltpu.async_copy(r_hbm.at[pl.ds(eoff, CHUNK)], rbuf[p], semrso[p])
        b = pltpu.async_copy(s_hbm.at[pl.ds(eoff, CHUNK)], sbuf[p], semrso[p])
        c = pltpu.async_copy(o_hbm.at[pl.ds(eoff, CHUNK)], obuf[p], semrso[p])
        return a, b, c

    def _wait_rso(p):
        pltpu.make_async_copy(r_hbm.at[pl.ds(0, CHUNK)], rbuf[p], semrso[p]).wait()
        pltpu.make_async_copy(s_hbm.at[pl.ds(0, CHUNK)], sbuf[p], semrso[p]).wait()
        pltpu.make_async_copy(o_hbm.at[pl.ds(0, CHUNK)], obuf[p], semrso[p]).wait()

    def _idx(j, p):
        inv = (cbase + j >= FWD_CHUNKS).astype(jnp.int32)
        for jj in range(SUB):
            r16 = lax.bitwise_and(rbuf[p][pl.ds(jj * L, L)], NREL - 1)
            sv = sbuf[p][pl.ds(jj * L, L)]
            ov = obuf[p][pl.ds(jj * L, L)]
            src = jnp.where(inv == 1, ov, sv)
            dst = jnp.where(inv == 1, sv, ov)
            rel = r16 + inv * NREL
            gidx[p][0, pl.ds(jj * L, L)] = rel * NNODES + src
            gidx[p][1, pl.ds(jj * L, L)] = rel * NNODES + dst
            dstb[p][0, pl.ds(jj * L, L)] = dst

    def _issue_gathers(p):
        pltpu.async_copy(counts_hbm.at[gidx[p].at[1]], cntb[p], semcnt[p])
        pltpu.async_copy(y_hbm.at[gidx[p].at[0]], rowb[p], semrow[p])

    def _wait_cnt(p):
        pltpu.make_async_copy(counts_hbm.at[gidx[p].at[1]], cntb[p], semcnt[p]).wait()

    def _wait_row(p):
        pltpu.make_async_copy(y_hbm.at[gidx[p].at[0]], rowb[p], semrow[p]).wait()

    def _issue_scatter(p):
        pltpu.async_copy(rowb[p], acc_sh.at[dstb[p].at[0]], semsca[p], add=True)

    def _wait_scatter(p):
        pltpu.make_async_copy(rowb[p], acc_sh.at[dstb[p].at[0]], semsca[p]).wait()

    def _norm(p):
        for jj in range(SUB):
            c = cntb[p][pl.ds(jj * L, L)]
            nrmb[p][0, pl.ds(jj * L, L)] = 1.0 / jnp.maximum(c, 1.0)

    def _scale(p):
        zeros16 = jnp.zeros((L,), jnp.int32)

        def _body(e, _):
            nv = plsc.load_gather(nrmb[p], [zeros16, jnp.full((L,), e, jnp.int32)])
            rowb[p][e, pl.ds(0, L)] = rowb[p][e, pl.ds(0, L)] * nv
            rowb[p][e, pl.ds(L, L)] = rowb[p][e, pl.ds(L, L)] * nv
            rowb[p][e, pl.ds(2 * L, L)] = rowb[p][e, pl.ds(2 * L, L)] * nv
            rowb[p][e, pl.ds(3 * L, L)] = rowb[p][e, pl.ds(3 * L, L)] * nv
            return 0
        lax.fori_loop(0, CHUNK, _body, 0)

    def _stage_ad(j, p):
        _wait_rso(p)

        @pl.when(j >= 2)
        def _():
            _wait_scatter(p)
        _idx(j, p)
        _issue_gathers(p)

        @pl.when(j + 2 < NCH)
        def _():
            _issue_rso(j + 2, p)

    def _stage_eg(j, p):
        _wait_cnt(p)
        _norm(p)
        _wait_row(p)
        _scale(p)
        _issue_scatter(p)

    _issue_rso(0, 0)
    _issue_rso(1, 1)
    _stage_ad(jnp.int32(0), 0)

    def _pair(i2, _):
        for p in range(2):
            i = 2 * i2 + p

            @pl.when(i < NCH)
            def _():
                @pl.when(i + 1 < NCH)
                def _():
                    _stage_ad(i + 1, 1 - p)
                _stage_eg(i, p)
        return 0
    lax.fori_loop(0, (NCH + 1) // 2, _pair, 0)

    _wait_scatter((NCH - 1) % 2)
    _wait_scatter((NCH - 2) % 2)
    plsc.subcore_barrier()

    def _out(i, _):
        pltpu.sync_copy(acc_sh.at[pl.ds(base_row + i * 80, 80)], zbuf)
        pltpu.sync_copy(zbuf, hpart_hbm.at[cid, pl.ds(base_row + i * 80, 80)])
        return 0
    lax.fori_loop(0, 7, _out, 0)
    pltpu.sync_copy(acc_sh.at[pl.ds(base_row + 560, 66)], zbuf.at[pl.ds(0, 66)])
    pltpu.sync_copy(zbuf.at[pl.ds(0, 66)], hpart_hbm.at[cid, pl.ds(base_row + 560, 66)])


def _sc_layer(yf, s_col, r_col, o_col, counts):
    k = pl.kernel(
        _k_layer_body,
        out_type=jax.ShapeDtypeStruct((2, ACC_ROWS, NEMB), jnp.float32),
        mesh=_MESH,
        compiler_params=_SC_PARAMS,
        scratch_types=(
            [pltpu.VMEM((CHUNK,), jnp.int32)] * 6
            + [pltpu.VMEM((2, CHUNK), jnp.int32)] * 2
            + [pltpu.VMEM((1, CHUNK), jnp.int32)] * 2
            + [pltpu.VMEM((CHUNK,), jnp.float32)] * 2
            + [pltpu.VMEM((1, CHUNK), jnp.float32)] * 2
            + [pltpu.VMEM((CHUNK, NEMB), jnp.float32)] * 2
            + [pltpu.VMEM((80, NEMB), jnp.float32)]
            + [pltpu.VMEM_SHARED((ACC_ROWS, NEMB), jnp.float32)]
            + [pltpu.SemaphoreType.DMA] * 8
        ),
    )
    return k(yf, s_col, r_col, o_col, counts)


def _k_distmult_body(h_hbm, rel_hbm, ts_hbm, tp_hbm, to_hbm, scores_hbm,
                     tsbuf, tpbuf, tobuf, idxbuf, hsrow, horow, relrow,
                     tmpbuf, scorebuf, sem):
    cid = lax.axis_index("c")
    sid = lax.axis_index("s")
    wid = cid * NS + sid
    per_w = N_TRIPLES // NW
    n_ch = per_w // CHUNK

    def _chunk(ci, _):
        toff = wid * per_w + ci * CHUNK
        a1 = pltpu.async_copy(ts_hbm.at[pl.ds(toff, CHUNK)], tsbuf, sem)
        a2 = pltpu.async_copy(tp_hbm.at[pl.ds(toff, CHUNK)], tpbuf, sem)
        a3 = pltpu.async_copy(to_hbm.at[pl.ds(toff, CHUNK)], tobuf, sem)
        a1.wait(); a2.wait(); a3.wait()

        for j in range(SUB):
            idxbuf[0, pl.ds(j * L, L)] = tsbuf[pl.ds(j * L, L)]
            idxbuf[1, pl.ds(j * L, L)] = lax.bitwise_and(tpbuf[pl.ds(j * L, L)], NREL - 1)
            idxbuf[2, pl.ds(j * L, L)] = tobuf[pl.ds(j * L, L)]

        g1 = pltpu.async_copy(h_hbm.at[idxbuf.at[0]], hsrow, sem)
        g2 = pltpu.async_copy(rel_hbm.at[idxbuf.at[1]], relrow, sem)
        g3 = pltpu.async_copy(h_hbm.at[idxbuf.at[2]], horow, sem)
        g1.wait(); g2.wait(); g3.wait()

        def _dot(e, _):
            acc = (hsrow[e, pl.ds(0, L)] * relrow[e, pl.ds(0, L)] * horow[e, pl.ds(0, L)]
                   + hsrow[e, pl.ds(L, L)] * relrow[e, pl.ds(L, L)] * horow[e, pl.ds(L, L)]
                   + hsrow[e, pl.ds(2 * L, L)] * relrow[e, pl.ds(2 * L, L)] * horow[e, pl.ds(2 * L, L)]
                   + hsrow[e, pl.ds(3 * L, L)] * relrow[e, pl.ds(3 * L, L)] * horow[e, pl.ds(3 * L, L)])
            tmpbuf[e, pl.ds(0, L)] = acc
            return 0
        lax.fori_loop(0, CHUNK, _dot, 0)

        rows16 = lax.iota(jnp.int32, L)

        def _red(j, _):
            ridx = rows16 + j * L
            sv = jnp.zeros((L,), jnp.float32)

            def _acc(k, sv):
                return sv + plsc.load_gather(tmpbuf, [ridx, jnp.full((L,), k, jnp.int32)])
            sv = lax.fori_loop(0, L, _acc, sv)
            scorebuf[pl.ds(j * L, L)] = sv
            return 0
        lax.fori_loop(0, SUB, _red, 0)

        pltpu.sync_copy(scorebuf, scores_hbm.at[pl.ds(toff, CHUNK)])
        return 0

    lax.fori_loop(0, n_ch, _chunk, 0)


def _sc_distmult(h2, relations, ts, tp, to):
    k = pl.kernel(
        _k_distmult_body,
        out_type=jax.ShapeDtypeStruct((N_TRIPLES,), jnp.float32),
        mesh=_MESH,
        compiler_params=_SC_PARAMS,
        scratch_types=[
            pltpu.VMEM((CHUNK,), jnp.int32),
            pltpu.VMEM((CHUNK,), jnp.int32),
            pltpu.VMEM((CHUNK,), jnp.int32),
            pltpu.VMEM((3, CHUNK), jnp.int32),
            pltpu.VMEM((CHUNK, NEMB), jnp.float32),
            pltpu.VMEM((CHUNK, NEMB), jnp.float32),
            pltpu.VMEM((CHUNK, NEMB), jnp.float32),
            pltpu.VMEM((CHUNK, L), jnp.float32),
            pltpu.VMEM((CHUNK,), jnp.float32),
            pltpu.SemaphoreType.DMA,
        ],
    )
    return k(h2, relations, ts, tp, to)


def kernel(node_embeddings, node_embeddings_bias, W1, b1, W2, b2, relations, graph, triples):
    graph = graph.astype(jnp.int32)
    triples = triples.astype(jnp.int32)
    npad = PAD_EDGES - N_EDGES
    s_col = jnp.concatenate([graph[:, 0], jnp.full((npad,), NNODES, jnp.int32)])
    r_col = jnp.concatenate([graph[:, 1], jnp.full((npad,), NREL - 1, jnp.int32)])
    o_col = jnp.concatenate([graph[:, 2], jnp.full((npad,), NNODES, jnp.int32)])
    ts = triples[:, 0]
    tp = triples[:, 1]
    to = triples[:, 2]

    y1, pen = _tc_layer1(node_embeddings, node_embeddings_bias, W1, relations)
    counts = _sc_counts(s_col, r_col, o_col)

    y1f = y1.reshape(R_TOTAL * NNODES, NEMB)
    hpart1 = _sc_layer(y1f, s_col, r_col, o_col, counts)

    y2 = _tc_layer2(hpart1, y1, b1, W2)
    y2f = y2.reshape(R_TOTAL * NNODES, NEMB)
    hpart2 = _sc_layer(y2f, s_col, r_col, o_col, counts)

    h2 = _tc_final(hpart2, y2, b2)
    scores = _sc_distmult(h2, relations, ts, tp, to)
    return (scores, pen[0, 0])


# speedup vs baseline: 11.9490x; 1.0824x over previous
import functools
import jax
import jax.numpy as jnp
from jax import lax
from jax.experimental import pallas as pl
from jax.experimental.pallas import tpu as pltpu
from jax.experimental.pallas import tpu_sc as plsc

NNODES = 10000
NREL = 16
NEMB = 64
R_TOTAL = 2 * NREL + 1
N_EDGES = 320000
N_TRIPLES = 16384

NC, NS, L = 2, 16, 16
NW = NC * NS
CHUNK = 128
PAD_EDGES = 327680
FWD_CHUNKS = PAD_EDGES // CHUNK
N_CHUNKS = 2 * FWD_CHUNKS
CPW = N_CHUNKS // NW
CPT = FWD_CHUNKS // NS
MAC = 16
NSEG = 2 * NREL * NNODES + 16
NSEG_HALF = NREL * NNODES
ACC_ROWS = NNODES + 16
SUB = CHUNK // L

_MESH = plsc.VectorSubcoreMesh(core_axis_name="c", subcore_axis_name="s")
_SC_PARAMS = pltpu.CompilerParams(use_tc_tiling_on_sc=False, needs_layout_passes=False)


def _k_a_body(x_ref, b_ref, w_ref, rel_ref, y_ref, pen_ref):
    x = jnp.maximum(x_ref[...] + b_ref[...], 0.0)
    y_ref[0] = jnp.dot(x, w_ref[0], preferred_element_type=jnp.float32)

    @pl.when(pl.program_id(0) == 0)
    def _():
        pen_ref[...] = jnp.sum(rel_ref[...] ** 2).reshape(1, 1)


def _tc_layer1(ne, bias, W1, relations):
    return pl.pallas_call(
        _k_a_body,
        grid=(R_TOTAL,),
        in_specs=[
            pl.BlockSpec((NNODES, NEMB), lambda r: (0, 0)),
            pl.BlockSpec((1, NEMB), lambda r: (0, 0)),
            pl.BlockSpec((1, NEMB, NEMB), lambda r: (r, 0, 0)),
            pl.BlockSpec((NREL, NEMB), lambda r: (0, 0)),
        ],
        out_specs=[
            pl.BlockSpec((1, NNODES, NEMB), lambda r: (r, 0, 0)),
            pl.BlockSpec((1, 1), lambda r: (0, 0)),
        ],
        out_shape=[
            jax.ShapeDtypeStruct((R_TOTAL, NNODES, NEMB), jnp.float32),
            jax.ShapeDtypeStruct((1, 1), jnp.float32),
        ],
    )(ne, bias, W1, relations)


def _k_e_body(p_ref, yself_ref, b_ref, w_ref, y_ref):
    x = p_ref[0] + p_ref[1] + yself_ref[0] + b_ref[...]
    x = jnp.maximum(x, 0.0)
    y_ref[0] = jnp.dot(x, w_ref[0], preferred_element_type=jnp.float32)


def _tc_layer2(hpart, y1, b1, W2):
    return pl.pallas_call(
        _k_e_body,
        grid=(R_TOTAL,),
        in_specs=[
            pl.BlockSpec((2, NNODES, NEMB), lambda r: (0, 0, 0)),
            pl.BlockSpec((1, NNODES, NEMB), lambda r: (R_TOTAL - 1, 0, 0)),
            pl.BlockSpec((1, NEMB), lambda r: (0, 0)),
            pl.BlockSpec((1, NEMB, NEMB), lambda r: (r, 0, 0)),
        ],
        out_specs=pl.BlockSpec((1, NNODES, NEMB), lambda r: (r, 0, 0)),
        out_shape=jax.ShapeDtypeStruct((R_TOTAL, NNODES, NEMB), jnp.float32),
    )(hpart, y1, b1.reshape(1, NEMB), W2)


def _k_f_body(p_ref, yself_ref, b_ref, h_ref):
    h_ref[...] = p_ref[0] + p_ref[1] + yself_ref[0] + b_ref[...]


def _tc_final(hpart, y2, b2):
    return pl.pallas_call(
        _k_f_body,
        grid=(1,),
        in_specs=[
            pl.BlockSpec((2, NNODES, NEMB), lambda i: (0, 0, 0)),
            pl.BlockSpec((1, NNODES, NEMB), lambda i: (R_TOTAL - 1, 0, 0)),
            pl.BlockSpec((1, NEMB), lambda i: (0, 0)),
        ],
        out_specs=pl.BlockSpec((NNODES, NEMB), lambda i: (0, 0)),
        out_shape=jax.ShapeDtypeStruct((NNODES, NEMB), jnp.float32),
    )(hpart, y2, b2.reshape(1, NEMB))


def _k_counts_body(s_hbm, r_hbm, o_hbm, norm_hbm,
                   rmac, dmac, segmac, onesbuf, zbuf, cbuf, nmac,
                   counts_sh, semL, semS):
    cid = lax.axis_index("c")
    sid = lax.axis_index("s")

    for i in range(SUB):
        onesbuf[0, pl.ds(i * L, L)] = jnp.full((L,), 1.0, jnp.float32)

    def _zero(i, _):
        zbuf[pl.ds(i * L, L)] = jnp.zeros((L,), jnp.float32)
        return 0
    lax.fori_loop(0, 2000 // L, _zero, 0)

    def _zslice(i, _):
        pltpu.sync_copy(zbuf, counts_sh.at[pl.ds(sid * 10000 + i * 2000, 2000)])
        return 0
    lax.fori_loop(0, 5, _zslice, 0)

    @pl.when(sid == 15)
    def _ztail():
        pltpu.sync_copy(zbuf.at[pl.ds(0, 16)], counts_sh.at[pl.ds(NSEG_HALF, 16)])
    plsc.subcore_barrier()

    def _macro(m, _):
        eoff = (sid * CPT + m * MAC) * CHUNK
        a = pltpu.async_copy(r_hbm.at[pl.ds(eoff, MAC * CHUNK)], rmac, semL)

        @pl.when(cid == 0)
        def _():
            pltpu.async_copy(o_hbm.at[pl.ds(eoff, MAC * CHUNK)], dmac, semL)

        @pl.when(cid == 1)
        def _():
            pltpu.async_copy(s_hbm.at[pl.ds(eoff, MAC * CHUNK)], dmac, semL)
        a.wait()
        pltpu.make_async_copy(o_hbm.at[pl.ds(0, MAC * CHUNK)], dmac, semL).wait()

        for k in range(MAC):
            for j in range(SUB):
                off = k * CHUNK + j * L
                rel = lax.bitwise_and(rmac[pl.ds(off, L)], NREL - 1)
                seg = rel * NNODES + dmac[pl.ds(off, L)]
                segmac[k, pl.ds(j * L, L)] = seg
            pltpu.async_copy(onesbuf.at[0], counts_sh.at[segmac.at[k]], semS, add=True)
        for k in range(MAC):
            pltpu.make_async_copy(onesbuf.at[0], counts_sh.at[segmac.at[k]], semS).wait()
        return 0
    lax.fori_loop(0, CPT // MAC, _macro, 0)

    plsc.subcore_barrier()

    def _macro2(m, _):
        eoff = (sid * CPT + m * MAC) * CHUNK
        a = pltpu.async_copy(r_hbm.at[pl.ds(eoff, MAC * CHUNK)], rmac, semL)

        @pl.when(cid == 0)
        def _():
            pltpu.async_copy(o_hbm.at[pl.ds(eoff, MAC * CHUNK)], dmac, semL)

        @pl.when(cid == 1)
        def _():
            pltpu.async_copy(s_hbm.at[pl.ds(eoff, MAC * CHUNK)], dmac, semL)
        a.wait()
        pltpu.make_async_copy(o_hbm.at[pl.ds(0, MAC * CHUNK)], dmac, semL).wait()

        for k in range(MAC):
            for j in range(SUB):
                off = k * CHUNK + j * L
                rel = lax.bitwise_and(rmac[pl.ds(off, L)], NREL - 1)
                seg = rel * NNODES + dmac[pl.ds(off, L)]
                segmac[k, pl.ds(j * L, L)] = seg
            pltpu.sync_copy(counts_sh.at[segmac.at[k]], cbuf)
            for j in range(SUB):
                c = cbuf[pl.ds(j * L, L)]
                nmac[pl.ds(k * CHUNK + j * L, L)] = 1.0 / jnp.maximum(c, 1.0)
        pltpu.sync_copy(nmac, norm_hbm.at[pl.ds(cid * PAD_EDGES + eoff, MAC * CHUNK)])
        return 0

    lax.fori_loop(0, CPT // MAC, _macro2, 0)


def _sc_counts(s_col, r_col, o_col):
    k = pl.kernel(
        _k_counts_body,
        out_type=jax.ShapeDtypeStruct((2 * PAD_EDGES,), jnp.float32),
        mesh=_MESH,
        compiler_params=_SC_PARAMS,
        scratch_types=[
            pltpu.VMEM((MAC * CHUNK,), jnp.int32),
            pltpu.VMEM((MAC * CHUNK,), jnp.int32),
            pltpu.VMEM((MAC, CHUNK), jnp.int32),
            pltpu.VMEM((1, CHUNK), jnp.float32),
            pltpu.VMEM((2000,), jnp.float32),
            pltpu.VMEM((CHUNK,), jnp.float32),
            pltpu.VMEM((MAC * CHUNK,), jnp.float32),
            pltpu.VMEM_SHARED((NSEG_HALF + 16,), jnp.float32),
            pltpu.SemaphoreType.DMA,
            pltpu.SemaphoreType.DMA,
        ],
    )
    return k(s_col, r_col, o_col)


def _k_layer_body(y_hbm, s_hbm, r_hbm, o_hbm, norm_hbm, hpart_hbm,
                  rbuf0, rbuf1, sbuf0, sbuf1, obuf0, obuf1,
                  gidx0, gidx1, dst0, dst1, nrm0, nrm1,
                  row0, row1, zbuf, acc_sh,
                  semrso0, semrso1, semnrm0, semnrm1,
                  semrow0, semrow1, semsca0, semsca1):
    cid = lax.axis_index("c")
    sid = lax.axis_index("s")
    wid = cid * NS + sid

    rbuf = (rbuf0, rbuf1)
    sbuf = (sbuf0, sbuf1)
    obuf = (obuf0, obuf1)
    gidx = (gidx0, gidx1)
    dstb = (dst0, dst1)
    nrmb = (nrm0, nrm1)
    rowb = (row0, row1)
    semrso = (semrso0, semrso1)
    semnrm = (semnrm0, semnrm1)
    semrow = (semrow0, semrow1)
    semsca = (semsca0, semsca1)

    def _zfill(i, _):
        r = i // 4
        c = i % 4
        zbuf[r, pl.ds(c * L, L)] = jnp.zeros((L,), jnp.float32)
        return 0
    lax.fori_loop(0, 320, _zfill, 0)

    base_row = sid * 626

    def _zslice(i, _):
        pltpu.sync_copy(zbuf, acc_sh.at[pl.ds(base_row + i * 80, 80)])
        return 0
    lax.fori_loop(0, 7, _zslice, 0)
    pltpu.sync_copy(zbuf.at[pl.ds(0, 66)], acc_sh.at[pl.ds(base_row + 560, 66)])
    plsc.subcore_barrier()

    cbase = wid * CPW
    NCH = CPW

    def _issue_rso(j, p):
        eoff = (lax.rem(cbase + j, FWD_CHUNKS)) * CHUNK
        a = pltpu.async_copy(r_hbm.at[pl.ds(eoff, CHUNK)], rbuf[p], semrso[p])
        b = pltpu.async_copy(s_hbm.at[pl.ds(eoff, CHUNK)], sbuf[p], semrso[p])
        c = pltpu.async_copy(o_hbm.at[pl.ds(eoff, CHUNK)], obuf[p], semrso[p])
        return a, b, c

    def _wait_rso(p):
        pltpu.make_async_copy(r_hbm.at[pl.ds(0, CHUNK)], rbuf[p], semrso[p]).wait()
        pltpu.make_async_copy(s_hbm.at[pl.ds(0, CHUNK)], sbuf[p], semrso[p]).wait()
        pltpu.make_async_copy(o_hbm.at[pl.ds(0, CHUNK)], obuf[p], semrso[p]).wait()

    def _idx(j, p):
        inv = (cbase + j >= FWD_CHUNKS).astype(jnp.int32)
        for jj in range(SUB):
            r16 = lax.bitwise_and(rbuf[p][pl.ds(jj * L, L)], NREL - 1)
            sv = sbuf[p][pl.ds(jj * L, L)]
            ov = obuf[p][pl.ds(jj * L, L)]
            src = jnp.where(inv == 1, ov, sv)
            dst = jnp.where(inv == 1, sv, ov)
            rel = r16 + inv * NREL
            gidx[p][0, pl.ds(jj * L, L)] = rel * NNODES + src
            dstb[p][0, pl.ds(jj * L, L)] = dst

    def _issue_gathers(j, p):
        noff = (cbase + j) * CHUNK
        pltpu.async_copy(norm_hbm.at[pl.ds(noff, CHUNK)], nrmb[p].at[0], semnrm[p])
        pltpu.async_copy(y_hbm.at[gidx[p].at[0]], rowb[p], semrow[p])

    def _wait_nrm(p):
        pltpu.make_async_copy(
            norm_hbm.at[pl.ds(0, CHUNK)], nrmb[p].at[0], semnrm[p]).wait()

    def _wait_row(p):
        pltpu.make_async_copy(y_hbm.at[gidx[p].at[0]], rowb[p], semrow[p]).wait()

    def _issue_scatter(p):
        pltpu.async_copy(rowb[p], acc_sh.at[dstb[p].at[0]], semsca[p], add=True)

    def _wait_scatter(p):
        pltpu.make_async_copy(rowb[p], acc_sh.at[dstb[p].at[0]], semsca[p]).wait()

    def _scale(p):
        zeros16 = jnp.zeros((L,), jnp.int32)

        def _body(e4, _):
            for u in range(4):
                e = e4 * 4 + u
                nv = plsc.load_gather(nrmb[p], [zeros16, jnp.full((L,), e, jnp.int32)])
                rowb[p][e, pl.ds(0, L)] = rowb[p][e, pl.ds(0, L)] * nv
                rowb[p][e, pl.ds(L, L)] = rowb[p][e, pl.ds(L, L)] * nv
                rowb[p][e, pl.ds(2 * L, L)] = rowb[p][e, pl.ds(2 * L, L)] * nv
                rowb[p][e, pl.ds(3 * L, L)] = rowb[p][e, pl.ds(3 * L, L)] * nv
            return 0
        lax.fori_loop(0, CHUNK // 4, _body, 0)

    def _stage_ad(j, p):
        _wait_rso(p)

        @pl.when(j >= 2)
        def _():
            _wait_scatter(p)
        _idx(j, p)
        _issue_gathers(j, p)

        @pl.when(j + 2 < NCH)
        def _():
            _issue_rso(j + 2, p)

    def _stage_eg(j, p):
        _wait_nrm(p)
        _wait_row(p)
        _scale(p)
        _issue_scatter(p)

    _issue_rso(0, 0)
    _issue_rso(1, 1)
    _stage_ad(jnp.int32(0), 0)

    def _pair(i2, _):
        for p in range(2):
            i = 2 * i2 + p

            @pl.when(i < NCH)
            def _():
                @pl.when(i + 1 < NCH)
                def _():
                    _stage_ad(i + 1, 1 - p)
                _stage_eg(i, p)
        return 0
    lax.fori_loop(0, (NCH + 1) // 2, _pair, 0)

    _wait_scatter((NCH - 1) % 2)
    _wait_scatter((NCH - 2) % 2)
    plsc.subcore_barrier()

    def _out(i, _):
        pltpu.sync_copy(acc_sh.at[pl.ds(base_row + i * 80, 80)], zbuf)
        pltpu.sync_copy(zbuf, hpart_hbm.at[cid, pl.ds(base_row + i * 80, 80)])
        return 0
    lax.fori_loop(0, 7, _out, 0)
    pltpu.sync_copy(acc_sh.at[pl.ds(base_row + 560, 66)], zbuf.at[pl.ds(0, 66)])
    pltpu.sync_copy(zbuf.at[pl.ds(0, 66)], hpart_hbm.at[cid, pl.ds(base_row + 560, 66)])


def _sc_layer(yf, s_col, r_col, o_col, norm):
    k = pl.kernel(
        _k_layer_body,
        out_type=jax.ShapeDtypeStruct((2, ACC_ROWS, NEMB), jnp.float32),
        mesh=_MESH,
        compiler_params=_SC_PARAMS,
        scratch_types=(
            [pltpu.VMEM((CHUNK,), jnp.int32)] * 6
            + [pltpu.VMEM((1, CHUNK), jnp.int32)] * 2
            + [pltpu.VMEM((1, CHUNK), jnp.int32)] * 2
            + [pltpu.VMEM((1, CHUNK), jnp.float32)] * 2
            + [pltpu.VMEM((CHUNK, NEMB), jnp.float32)] * 2
            + [pltpu.VMEM((80, NEMB), jnp.float32)]
            + [pltpu.VMEM_SHARED((ACC_ROWS, NEMB), jnp.float32)]
            + [pltpu.SemaphoreType.DMA] * 8
        ),
    )
    return k(yf, s_col, r_col, o_col, norm)


def _k_distmult_body(h_hbm, rel_hbm, ts_hbm, tp_hbm, to_hbm, scores_hbm,
                     tsbuf, tpbuf, tobuf, idxbuf, hsrow, horow, relrow,
                     tmpbuf, scorebuf, sem):
    cid = lax.axis_index("c")
    sid = lax.axis_index("s")
    wid = cid * NS + sid
    per_w = N_TRIPLES // NW
    n_ch = per_w // CHUNK

    def _chunk(ci, _):
        toff = wid * per_w + ci * CHUNK
        a1 = pltpu.async_copy(ts_hbm.at[pl.ds(toff, CHUNK)], tsbuf, sem)
        a2 = pltpu.async_copy(tp_hbm.at[pl.ds(toff, CHUNK)], tpbuf, sem)
        a3 = pltpu.async_copy(to_hbm.at[pl.ds(toff, CHUNK)], tobuf, sem)
        a1.wait(); a2.wait(); a3.wait()

        for j in range(SUB):
            idxbuf[0, pl.ds(j * L, L)] = tsbuf[pl.ds(j * L, L)]
            idxbuf[1, pl.ds(j * L, L)] = lax.bitwise_and(tpbuf[pl.ds(j * L, L)], NREL - 1)
            idxbuf[2, pl.ds(j * L, L)] = tobuf[pl.ds(j * L, L)]

        g1 = pltpu.async_copy(h_hbm.at[idxbuf.at[0]], hsrow, sem)
        g2 = pltpu.async_copy(rel_hbm.at[idxbuf.at[1]], relrow, sem)
        g3 = pltpu.async_copy(h_hbm.at[idxbuf.at[2]], horow, sem)
        g1.wait(); g2.wait(); g3.wait()

        def _dot(e, _):
            acc = (hsrow[e, pl.ds(0, L)] * relrow[e, pl.ds(0, L)] * horow[e, pl.ds(0, L)]
                   + hsrow[e, pl.ds(L, L)] * relrow[e, pl.ds(L, L)] * horow[e, pl.ds(L, L)]
                   + hsrow[e, pl.ds(2 * L, L)] * relrow[e, pl.ds(2 * L, L)] * horow[e, pl.ds(2 * L, L)]
                   + hsrow[e, pl.ds(3 * L, L)] * relrow[e, pl.ds(3 * L, L)] * horow[e, pl.ds(3 * L, L)])
            tmpbuf[e, pl.ds(0, L)] = acc
            return 0
        lax.fori_loop(0, CHUNK, _dot, 0)

        rows16 = lax.iota(jnp.int32, L)

        def _red(j, _):
            ridx = rows16 + j * L
            sv = jnp.zeros((L,), jnp.float32)

            def _acc(k, sv):
                return sv + plsc.load_gather(tmpbuf, [ridx, jnp.full((L,), k, jnp.int32)])
            sv = lax.fori_loop(0, L, _acc, sv)
            scorebuf[pl.ds(j * L, L)] = sv
            return 0
        lax.fori_loop(0, SUB, _red, 0)

        pltpu.sync_copy(scorebuf, scores_hbm.at[pl.ds(toff, CHUNK)])
        return 0

    lax.fori_loop(0, n_ch, _chunk, 0)


def _sc_distmult(h2, relations, ts, tp, to):
    k = pl.kernel(
        _k_distmult_body,
        out_type=jax.ShapeDtypeStruct((N_TRIPLES,), jnp.float32),
        mesh=_MESH,
        compiler_params=_SC_PARAMS,
        scratch_types=[
            pltpu.VMEM((CHUNK,), jnp.int32),
            pltpu.VMEM((CHUNK,), jnp.int32),
            pltpu.VMEM((CHUNK,), jnp.int32),
            pltpu.VMEM((3, CHUNK), jnp.int32),
            pltpu.VMEM((CHUNK, NEMB), jnp.float32),
            pltpu.VMEM((CHUNK, NEMB), jnp.float32),
            pltpu.VMEM((CHUNK, NEMB), jnp.float32),
            pltpu.VMEM((CHUNK, L), jnp.float32),
            pltpu.VMEM((CHUNK,), jnp.float32),
            pltpu.SemaphoreType.DMA,
        ],
    )
    return k(h2, relations, ts, tp, to)


def kernel(node_embeddings, node_embeddings_bias, W1, b1, W2, b2, relations, graph, triples):
    graph = graph.astype(jnp.int32)
    triples = triples.astype(jnp.int32)
    npad = PAD_EDGES - N_EDGES
    s_col = jnp.concatenate([graph[:, 0], jnp.full((npad,), NNODES, jnp.int32)])
    r_col = jnp.concatenate([graph[:, 1], jnp.full((npad,), NREL - 1, jnp.int32)])
    o_col = jnp.concatenate([graph[:, 2], jnp.full((npad,), NNODES, jnp.int32)])
    ts = triples[:, 0]
    tp = triples[:, 1]
    to = triples[:, 2]

    y1, pen = _tc_layer1(node_embeddings, node_embeddings_bias, W1, relations)
    norm = _sc_counts(s_col, r_col, o_col)

    y1f = y1.reshape(R_TOTAL * NNODES, NEMB)
    hpart1 = _sc_layer(y1f, s_col, r_col, o_col, norm)

    y2 = _tc_layer2(hpart1, y1, b1, W2)
    y2f = y2.reshape(R_TOTAL * NNODES, NEMB)
    hpart2 = _sc_layer(y2f, s_col, r_col, o_col, norm)

    h2 = _tc_final(hpart2, y2, b2)
    scores = _sc_distmult(h2, relations, ts, tp, to)
    return (scores, pen[0, 0])
